# Initial kernel scaffold; baseline (speedup 1.0000x reference)
import jax
import jax.numpy as jnp
from jax.experimental import pallas as pl


def kernel(x, edge_index, params):
    raise NotImplementedError("write your pallas kernel here")


# speedup vs baseline: 3.5106x; 3.5106x over previous
import functools

import jax
import jax.numpy as jnp
from jax import lax
from jax.experimental import pallas as pl
from jax.experimental.pallas import tpu as pltpu
from jax.experimental.pallas import tpu_sc as plsc

_N = 10000
_D = 128
_E = 320000

_NC = 2
_NS = 16
_NW = _NC * _NS
_NPAD = 10240
_ROWS_PER_TILE = _NPAD // _NS
_CHUNK = 128
_NCHUNKS = 79
_EPW = _NCHUNKS * _CHUNK
_EPAD = _EPW * _NW

@functools.cache
def _get_sc_segment_sum():
    mesh = plsc.VectorSubcoreMesh(
        core_axis_name="c", subcore_axis_name="s",
        num_cores=_NC, num_subcores=_NS)
    return functools.partial(
        pl.kernel,
        mesh=mesh,
        out_type=jax.ShapeDtypeStruct((_NC, _NPAD, _D), jnp.float32),
        scratch_types=[
            pltpu.VMEM((_CHUNK,), jnp.int32),
            pltpu.VMEM((_CHUNK,), jnp.int32),
            pltpu.VMEM((_CHUNK, _D), jnp.float32),
            pltpu.VMEM_SHARED((_NPAD, _D), jnp.float32),
            pltpu.SemaphoreType.DMA,
        ],
    )(_sc_segment_sum_body)


def _sc_segment_sum_body(src_hbm, dst_hbm, h_hbm, zeros_hbm, out_hbm,
                         srcbuf, dstbuf, rows, acc, sem):
    cid = lax.axis_index("c")
    sid = lax.axis_index("s")
    wid = sid * _NC + cid
    r0 = sid * _ROWS_PER_TILE

    pltpu.sync_copy(zeros_hbm.at[pl.ds(r0, _ROWS_PER_TILE)],
                    acc.at[pl.ds(r0, _ROWS_PER_TILE)])
    plsc.subcore_barrier()

    base = wid * _EPW

    def body(c, carry):
        off = base + c * _CHUNK
        pltpu.sync_copy(src_hbm.at[pl.ds(off, _CHUNK)], srcbuf)
        pltpu.sync_copy(dst_hbm.at[pl.ds(off, _CHUNK)], dstbuf)
        pltpu.async_copy(h_hbm.at[srcbuf], rows, sem).wait()
        pltpu.sync_copy(rows, acc.at[dstbuf], add=True)
        return carry

    lax.fori_loop(0, _NCHUNKS, body, 0)
    plsc.subcore_barrier()

    pltpu.sync_copy(acc.at[pl.ds(r0, _ROWS_PER_TILE)],
                    out_hbm.at[cid, pl.ds(r0, _ROWS_PER_TILE), :])


def _bn(z, valid, g, b):
    zm = jnp.where(valid, z, 0.0)
    mean = jnp.sum(zm, axis=0, keepdims=True) * (1.0 / _N)
    var = jnp.sum(zm * zm, axis=0, keepdims=True) * (1.0 / _N) - mean * mean
    return (z - mean) * lax.rsqrt(var + 1e-5) * g + b


def _make_mlp(trailing_bn):
    def body(h_ref, p_ref, w1_ref, b1_ref, g1_ref, be1_ref, w2_ref, b2_ref,
             *rest):
        if trailing_bn:
            bng_ref, bnb_ref, out_ref = rest
        else:
            (out_ref,) = rest
        valid = lax.broadcasted_iota(jnp.int32, (_NPAD, 1), 0) < _N
        a = h_ref[...] + p_ref[0] + p_ref[1]
        a = jnp.where(valid, a, 0.0)
        z = jnp.dot(a, w1_ref[...], preferred_element_type=jnp.float32)
        z = z + b1_ref[...]
        z = _bn(z, valid, g1_ref[...], be1_ref[...])
        z = jnp.maximum(z, 0.0)
        z = jnp.dot(z, w2_ref[...], preferred_element_type=jnp.float32)
        z = z + b2_ref[...]
        if trailing_bn:
            z = _bn(z, valid, bng_ref[...], bnb_ref[...])
            z = jnp.maximum(z, 0.0)
        out_ref[...] = jnp.where(valid, z, 0.0)

    return pl.pallas_call(
        body,
        out_shape=jax.ShapeDtypeStruct((_NPAD, _D), jnp.float32),
    )


_mlp_mid = _make_mlp(True)
_mlp_final = _make_mlp(False)


def kernel(x, edge_index, params):
    pad = _EPAD - _E
    src = jnp.concatenate(
        [edge_index[0], jnp.full((pad,), _NPAD - 1, jnp.int32)])
    dst = jnp.concatenate(
        [edge_index[1], jnp.full((pad,), _NPAD - 1, jnp.int32)])
    zeros = jnp.zeros((_NPAD, _D), jnp.float32)
    h = jnp.zeros((_NPAD, _D), jnp.float32).at[:_N].set(x)

    sc_segment_sum = _get_sc_segment_sum()
    for i in (1, 2, 3):
        parts = sc_segment_sum(src, dst, h, zeros)
        args = (h, parts,
                params[f'W{i}_1'], params[f'b{i}_1'],
                params[f'mlp_g{i}'], params[f'mlp_b{i}'],
                params[f'W{i}_2'], params[f'b{i}_2'])
        if i < 3:
            h = _mlp_mid(*args, params[f'bn_g{i}'], params[f'bn_b{i}'])
        else:
            h = _mlp_final(*args)
    return h[:_N]
